# baseline (speedup 1.0000x reference)
import functools

import jax
import jax.numpy as jnp
from jax import lax
from jax.experimental import pallas as pl
from jax.experimental.pallas import tpu as pltpu
from jax.experimental.pallas import tpu_sc as plsc

B, S, D = 2, 4096, 64
BS = 64
NB = S // BS
M_EMB = 2 * S - 1
M_PAD = 2 * S
MW = M_PAD // 2
N_BLK = B * NB * NB
N_TASKS = BS * (B * NB // 2)

SBLK = 256
WBLK = 1024

_sc = plsc.get_sparse_core_info()
NC, NS = _sc.num_cores, _sc.num_subcores
NW = NC * NS
TASKS_PER_W = N_TASKS // NW


def _mm_body(q_ref, ee_ref, eo_ref, o_ref):
    se = lax.dot_general(
        q_ref[0], ee_ref[0], (((1,), (1,)), ((), ())),
        preferred_element_type=jnp.float32)
    so = lax.dot_general(
        q_ref[0], eo_ref[0], (((1,), (1,)), ((), ())),
        preferred_element_type=jnp.float32)
    we = lax.convert_element_type(
        lax.bitcast_convert_type(se.astype(jnp.bfloat16), jnp.uint16),
        jnp.uint32)
    wo = lax.convert_element_type(
        lax.bitcast_convert_type(so.astype(jnp.bfloat16), jnp.uint16),
        jnp.uint32)
    o_ref[0] = ((wo << 16) | we).astype(jnp.int32)


def _scores_packed(q, emb_e, emb_o):
    return pl.pallas_call(
        _mm_body,
        grid=(B, S // SBLK, MW // WBLK),
        in_specs=[
            pl.BlockSpec((1, SBLK, D), lambda b, s, w: (b, s, 0)),
            pl.BlockSpec((1, WBLK, D), lambda b, s, w: (b, w, 0)),
            pl.BlockSpec((1, WBLK, D), lambda b, s, w: (b, w, 0)),
        ],
        out_specs=pl.BlockSpec((1, SBLK, WBLK), lambda b, s, w: (b, s, w)),
        out_shape=jax.ShapeDtypeStruct((B, S, MW), jnp.int32),
        compiler_params=pltpu.CompilerParams(
            dimension_semantics=("parallel", "parallel", "parallel")),
    )(q, emb_e, emb_o)


@functools.partial(
    pl.kernel,
    mesh=plsc.VectorSubcoreMesh(core_axis_name="c", subcore_axis_name="s"),
    out_type=jax.ShapeDtypeStruct((BS * BS, N_BLK), jnp.float32),
    scratch_types=[
        pltpu.VMEM((BS, 2 * BS), jnp.int32),
        pltpu.VMEM((BS, 2 * BS), jnp.int32),
        pltpu.VMEM((2 * MW,), jnp.int32),
        pltpu.VMEM((2 * MW,), jnp.int32),
        pltpu.VMEM((BS, 2 * BS), jnp.float32),
        pltpu.VMEM((BS, 2 * BS), jnp.float32),
        pltpu.SemaphoreType.DMA,
        pltpu.SemaphoreType.DMA,
        pltpu.SemaphoreType.DMA,
        pltpu.SemaphoreType.DMA,
    ],
    compiler_params=pltpu.CompilerParams(needs_layout_passes=False),
)
def _gather_kernel(scores_hbm, info_hbm, out_hbm,
                   info0, info1, rows0, rows1, out0, out1,
                   si0, si1, so0, so1):
    wid = lax.axis_index("s") * NC + lax.axis_index("c")
    t0 = wid * TASKS_PER_W
    slots = ((info0, rows0, out0, si0, so0),
             (info1, rows1, out1, si1, so1))

    def issue_in(t, s):
        inf, rows, _, si, _ = slots[s]
        brp = t // BS
        i = lax.rem(t, BS)
        pltpu.async_copy(
            info_hbm.at[pl.ds(i * BS, BS), pl.ds(brp * 2 * BS, 2 * BS)],
            inf, si)
        pltpu.async_copy(scores_hbm.at[brp * 2 * BS + i],
                         rows.at[pl.ds(0, MW)], si)
        pltpu.async_copy(scores_hbm.at[brp * 2 * BS + BS + i],
                         rows.at[pl.ds(MW, MW)], si)

    def wait_in(s):
        inf, rows, _, si, _ = slots[s]
        pltpu.make_async_copy(
            info_hbm.at[pl.ds(0, BS), pl.ds(0, 2 * BS)], inf, si).wait()
        pltpu.make_async_copy(scores_hbm.at[0], rows.at[pl.ds(0, MW)],
                              si).wait()
        pltpu.make_async_copy(scores_hbm.at[0], rows.at[pl.ds(MW, MW)],
                              si).wait()

    def compute(s):
        inf, rows, out, _, _ = slots[s]

        def do_j(j2, carry):
            for dj in range(2):
                j = 2 * j2 + dj
                for k in range(2 * BS // 16):
                    iv = inf[j, pl.ds(k * 16, 16)]
                    col = lax.shift_right_logical(iv, 1)
                    if k >= BS // 16:
                        col = col + MW
                    w = plsc.load_gather(rows, [col])
                    sh = lax.shift_left(iv & 1, 4)
                    bits = lax.shift_left(
                        lax.shift_right_logical(w, sh), 16)
                    out[j, pl.ds(k * 16, 16)] = plsc.bitcast(
                        bits, jnp.float32)
            return carry

        lax.fori_loop(0, BS // 2, do_j, 0)

    def issue_out(t, s):
        _, _, out, _, so = slots[s]
        brp = t // BS
        i = lax.rem(t, BS)
        pltpu.async_copy(
            out,
            out_hbm.at[pl.ds(i * BS, BS), pl.ds(brp * 2 * BS, 2 * BS)], so)

    def wait_out(s):
        _, _, out, _, so = slots[s]
        pltpu.make_async_copy(
            out, out_hbm.at[pl.ds(0, BS), pl.ds(0, 2 * BS)], so).wait()

    issue_in(t0, 0)
    issue_in(t0 + 1, 1)

    def body(t2, carry):
        t = t0 + 2 * t2
        for s in range(2):
            wait_in(s)

            @pl.when(t2 > 0)
            def _():
                wait_out(s)

            compute(s)
            issue_out(t + s, s)

            @pl.when(t2 < TASKS_PER_W // 2 - 1)
            def _():
                issue_in(t + 2 + s, s)
        return carry

    lax.fori_loop(0, TASKS_PER_W // 2, body, 0)
    wait_out(0)
    wait_out(1)


def kernel(q, emb, info, sparsity_layout):
    del sparsity_layout
    emb_p = jnp.pad(emb, ((0, 0), (0, M_PAD - M_EMB), (0, 0)))
    emb16 = emb_p.astype(jnp.bfloat16)
    scores = _scores_packed(q.astype(jnp.bfloat16),
                            emb16[:, 0::2], emb16[:, 1::2])
    info_v = info.transpose(1, 2, 0).reshape(BS * BS, N_BLK)
    out_v = _gather_kernel(scores.reshape(B * S, MW), info_v)
    return out_v.reshape(BS, BS, N_BLK).transpose(2, 0, 1)


# speedup vs baseline: 1.3236x; 1.3236x over previous
import functools

import jax
import jax.numpy as jnp
from jax import lax
from jax.experimental import pallas as pl
from jax.experimental.pallas import tpu as pltpu
from jax.experimental.pallas import tpu_sc as plsc

B, S, D = 2, 4096, 64
BS = 64
NB = S // BS
M_EMB = 2 * S - 1
M_PAD = 2 * S
N_BLK = B * NB * NB
N_TASKS = BS * (B * NB // 2)

SBLK = 512
MBLK = 4096

_sc = plsc.get_sparse_core_info()
NC, NS = _sc.num_cores, _sc.num_subcores
NW = NC * NS
TASKS_PER_W = N_TASKS // NW


def _mm_body(q_ref, e_ref, o_ref):
    o_ref[0] = lax.dot_general(
        q_ref[0], e_ref[0], (((1,), (1,)), ((), ())),
        preferred_element_type=jnp.float32)


def _scores(q, emb_p):
    return pl.pallas_call(
        _mm_body,
        grid=(B, S // SBLK, M_PAD // MBLK),
        in_specs=[
            pl.BlockSpec((1, SBLK, D), lambda b, s, m: (b, s, 0)),
            pl.BlockSpec((1, MBLK, D), lambda b, s, m: (b, m, 0)),
        ],
        out_specs=pl.BlockSpec((1, SBLK, MBLK), lambda b, s, m: (b, s, m)),
        out_shape=jax.ShapeDtypeStruct((B, S, M_PAD), jnp.float32),
        compiler_params=pltpu.CompilerParams(
            dimension_semantics=("parallel", "parallel", "parallel")),
    )(q, emb_p)


@functools.partial(
    pl.kernel,
    mesh=plsc.VectorSubcoreMesh(core_axis_name="c", subcore_axis_name="s"),
    out_type=jax.ShapeDtypeStruct((BS * BS, N_BLK), jnp.float32),
    scratch_types=[
        pltpu.VMEM((BS, 2 * BS), jnp.int32),
        pltpu.VMEM((BS, 2 * BS), jnp.int32),
        pltpu.VMEM((2 * M_PAD,), jnp.float32),
        pltpu.VMEM((2 * M_PAD,), jnp.float32),
        pltpu.VMEM((BS, 2 * BS), jnp.float32),
        pltpu.VMEM((BS, 2 * BS), jnp.float32),
        pltpu.SemaphoreType.DMA,
        pltpu.SemaphoreType.DMA,
        pltpu.SemaphoreType.DMA,
        pltpu.SemaphoreType.DMA,
    ],
    compiler_params=pltpu.CompilerParams(needs_layout_passes=False),
)
def _gather_kernel(scores_hbm, info_hbm, out_hbm,
                   info0, info1, rows0, rows1, out0, out1,
                   si0, si1, so0, so1):
    wid = lax.axis_index("s") * NC + lax.axis_index("c")
    t0 = wid * TASKS_PER_W
    slots = ((info0, rows0, out0, si0, so0),
             (info1, rows1, out1, si1, so1))

    def issue_in(t, s):
        inf, rows, _, si, _ = slots[s]
        brp = t // BS
        i = lax.rem(t, BS)
        pltpu.async_copy(
            info_hbm.at[pl.ds(i * BS, BS), pl.ds(brp * 2 * BS, 2 * BS)],
            inf, si)
        pltpu.async_copy(scores_hbm.at[brp * 2 * BS + i],
                         rows.at[pl.ds(0, M_PAD)], si)
        pltpu.async_copy(scores_hbm.at[brp * 2 * BS + BS + i],
                         rows.at[pl.ds(M_PAD, M_PAD)], si)

    def wait_in(s):
        inf, rows, _, si, _ = slots[s]
        pltpu.make_async_copy(
            info_hbm.at[pl.ds(0, BS), pl.ds(0, 2 * BS)], inf, si).wait()
        pltpu.make_async_copy(scores_hbm.at[0], rows.at[pl.ds(0, M_PAD)],
                              si).wait()
        pltpu.make_async_copy(scores_hbm.at[0], rows.at[pl.ds(M_PAD, M_PAD)],
                              si).wait()

    def compute(s):
        inf, rows, out, _, _ = slots[s]

        def do_j(j, carry):
            for k in range(2 * BS // 16):
                iv = inf[j, pl.ds(k * 16, 16)]
                if k >= BS // 16:
                    iv = iv + M_PAD
                out[j, pl.ds(k * 16, 16)] = plsc.load_gather(rows, [iv])
            return carry

        lax.fori_loop(0, BS, do_j, 0)

    def issue_out(t, s):
        _, _, out, _, so = slots[s]
        brp = t // BS
        i = lax.rem(t, BS)
        pltpu.async_copy(
            out,
            out_hbm.at[pl.ds(i * BS, BS), pl.ds(brp * 2 * BS, 2 * BS)], so)

    def wait_out(s):
        _, _, out, _, so = slots[s]
        pltpu.make_async_copy(
            out, out_hbm.at[pl.ds(0, BS), pl.ds(0, 2 * BS)], so).wait()

    issue_in(t0, 0)
    issue_in(t0 + 1, 1)

    def body(t2, carry):
        t = t0 + 2 * t2
        for s in range(2):
            wait_in(s)

            @pl.when(t2 > 0)
            def _():
                wait_out(s)

            compute(s)
            issue_out(t + s, s)

            @pl.when(t2 < TASKS_PER_W // 2 - 1)
            def _():
                issue_in(t + 2 + s, s)
        return carry

    lax.fori_loop(0, TASKS_PER_W // 2, body, 0)
    wait_out(0)
    wait_out(1)


def kernel(q, emb, info, sparsity_layout):
    del sparsity_layout
    emb_p = jnp.pad(emb, ((0, 0), (0, M_PAD - M_EMB), (0, 0)))
    scores = _scores(q.astype(jnp.bfloat16),
                     emb_p.astype(jnp.bfloat16)).reshape(B * S, M_PAD)
    info_v = info.transpose(1, 2, 0).reshape(BS * BS, N_BLK)
    out_v = _gather_kernel(scores, info_v)
    return out_v.reshape(BS, BS, N_BLK).transpose(2, 0, 1)


# speedup vs baseline: 1.3581x; 1.0260x over previous
import functools

import jax
import jax.numpy as jnp
from jax import lax
from jax.experimental import pallas as pl
from jax.experimental.pallas import tpu as pltpu
from jax.experimental.pallas import tpu_sc as plsc

B, S, D = 2, 4096, 64
BS = 64
NB = S // BS
M_EMB = 2 * S - 1
M_PAD = 2 * S
N_BLK = B * NB * NB
N_TASKS = BS * (B * NB // 2)

SBLK = 512
MBLK = 8192

_sc = plsc.get_sparse_core_info()
NC, NS = _sc.num_cores, _sc.num_subcores
NW = NC * NS
TASKS_PER_W = N_TASKS // NW


def _mm_body(q_ref, e_ref, o_ref):
    o_ref[0] = lax.dot_general(
        q_ref[0], e_ref[0], (((1,), (1,)), ((), ())),
        preferred_element_type=jnp.float32)


def _scores(q, emb_p):
    return pl.pallas_call(
        _mm_body,
        grid=(B, S // SBLK, M_PAD // MBLK),
        in_specs=[
            pl.BlockSpec((1, SBLK, D), lambda b, s, m: (b, s, 0)),
            pl.BlockSpec((1, MBLK, D), lambda b, s, m: (b, m, 0)),
        ],
        out_specs=pl.BlockSpec((1, SBLK, MBLK), lambda b, s, m: (b, s, m)),
        out_shape=jax.ShapeDtypeStruct((B, S, M_PAD), jnp.float32),
        compiler_params=pltpu.CompilerParams(
            dimension_semantics=("parallel", "parallel", "parallel")),
    )(q, emb_p)


@functools.partial(
    pl.kernel,
    mesh=plsc.VectorSubcoreMesh(core_axis_name="c", subcore_axis_name="s"),
    out_type=jax.ShapeDtypeStruct((BS * BS, N_BLK), jnp.float32),
    scratch_types=[
        pltpu.VMEM((BS, 2 * BS), jnp.int32),
        pltpu.VMEM((BS, 2 * BS), jnp.int32),
        pltpu.VMEM((2 * M_PAD,), jnp.float32),
        pltpu.VMEM((2 * M_PAD,), jnp.float32),
        pltpu.VMEM((BS, 2 * BS), jnp.float32),
        pltpu.VMEM((BS, 2 * BS), jnp.float32),
        pltpu.SemaphoreType.DMA,
        pltpu.SemaphoreType.DMA,
        pltpu.SemaphoreType.DMA,
        pltpu.SemaphoreType.DMA,
    ],
    compiler_params=pltpu.CompilerParams(needs_layout_passes=False),
)
def _gather_kernel(scores_hbm, info_hbm, out_hbm,
                   info0, info1, rows0, rows1, out0, out1,
                   si0, si1, so0, so1):
    wid = lax.axis_index("s") * NC + lax.axis_index("c")
    t0 = wid * TASKS_PER_W
    slots = ((info0, rows0, out0, si0, so0),
             (info1, rows1, out1, si1, so1))

    def issue_in(t, s):
        inf, rows, _, si, _ = slots[s]
        brp = t // BS
        i = lax.rem(t, BS)
        pltpu.async_copy(
            info_hbm.at[pl.ds(i * BS, BS), pl.ds(brp * 2 * BS, 2 * BS)],
            inf, si)
        pltpu.async_copy(scores_hbm.at[brp * 2 * BS + i],
                         rows.at[pl.ds(0, M_PAD)], si)
        pltpu.async_copy(scores_hbm.at[brp * 2 * BS + BS + i],
                         rows.at[pl.ds(M_PAD, M_PAD)], si)

    def wait_in(s):
        inf, rows, _, si, _ = slots[s]
        pltpu.make_async_copy(
            info_hbm.at[pl.ds(0, BS), pl.ds(0, 2 * BS)], inf, si).wait()
        pltpu.make_async_copy(scores_hbm.at[0], rows.at[pl.ds(0, M_PAD)],
                              si).wait()
        pltpu.make_async_copy(scores_hbm.at[0], rows.at[pl.ds(M_PAD, M_PAD)],
                              si).wait()

    def compute(s):
        inf, rows, out, _, _ = slots[s]

        def do_j(j, carry):
            for k in range(2 * BS // 16):
                iv = inf[j, pl.ds(k * 16, 16)]
                if k >= BS // 16:
                    iv = iv + M_PAD
                out[j, pl.ds(k * 16, 16)] = plsc.load_gather(rows, [iv])
            return carry

        lax.fori_loop(0, BS, do_j, 0)

    def issue_out(t, s):
        _, _, out, _, so = slots[s]
        brp = t // BS
        i = lax.rem(t, BS)
        pltpu.async_copy(
            out,
            out_hbm.at[pl.ds(i * BS, BS), pl.ds(brp * 2 * BS, 2 * BS)], so)

    def wait_out(s):
        _, _, out, _, so = slots[s]
        pltpu.make_async_copy(
            out, out_hbm.at[pl.ds(0, BS), pl.ds(0, 2 * BS)], so).wait()

    issue_in(t0, 0)
    issue_in(t0 + 1, 1)

    def body(t2, carry):
        t = t0 + 2 * t2
        for s in range(2):
            wait_in(s)

            @pl.when(t2 > 0)
            def _():
                wait_out(s)

            compute(s)
            issue_out(t + s, s)

            @pl.when(t2 < TASKS_PER_W // 2 - 1)
            def _():
                issue_in(t + 2 + s, s)
        return carry

    lax.fori_loop(0, TASKS_PER_W // 2, body, 0)
    wait_out(0)
    wait_out(1)


def kernel(q, emb, info, sparsity_layout):
    del sparsity_layout
    emb_p = jnp.pad(emb, ((0, 0), (0, M_PAD - M_EMB), (0, 0)))
    scores = _scores(q.astype(jnp.bfloat16),
                     emb_p.astype(jnp.bfloat16)).reshape(B * S, M_PAD)
    info_v = info.transpose(1, 2, 0).reshape(BS * BS, N_BLK)
    out_v = _gather_kernel(scores, info_v)
    return out_v.reshape(BS, BS, N_BLK).transpose(2, 0, 1)


# speedup vs baseline: 1.7698x; 1.3032x over previous
import functools

import jax
import jax.numpy as jnp
from jax import lax
from jax.experimental import pallas as pl
from jax.experimental.pallas import tpu as pltpu
from jax.experimental.pallas import tpu_sc as plsc

B, S, D = 2, 4096, 64
BS = 64
NB = S // BS
M_EMB = 2 * S - 1
M_PAD = 2 * S
N_BLK = B * NB * NB
N_TASKS = BS * (B * NB // 2)

SBLK = 512
MBLK = 8192

_sc = plsc.get_sparse_core_info()
NC, NS = _sc.num_cores, _sc.num_subcores
NW = NC * NS
TASKS_PER_W = N_TASKS // NW


def _mm_body(q_ref, e_ref, o_ref):
    o_ref[0] = lax.dot_general(
        q_ref[0], e_ref[0], (((1,), (1,)), ((), ())),
        preferred_element_type=jnp.float32)


def _scores(q, emb_p):
    return pl.pallas_call(
        _mm_body,
        grid=(B, S // SBLK, M_PAD // MBLK),
        in_specs=[
            pl.BlockSpec((1, SBLK, D), lambda b, s, m: (b, s, 0)),
            pl.BlockSpec((1, MBLK, D), lambda b, s, m: (b, m, 0)),
        ],
        out_specs=pl.BlockSpec((1, SBLK, MBLK), lambda b, s, m: (b, s, m)),
        out_shape=jax.ShapeDtypeStruct((B, S, M_PAD), jnp.float32),
        compiler_params=pltpu.CompilerParams(
            dimension_semantics=("parallel", "parallel", "parallel")),
    )(q, emb_p)


@functools.partial(
    pl.kernel,
    mesh=plsc.VectorSubcoreMesh(core_axis_name="c", subcore_axis_name="s"),
    out_type=jax.ShapeDtypeStruct((BS * BS, N_BLK), jnp.float32),
    scratch_types=[
        pltpu.VMEM((BS, 2 * BS), jnp.int32),
        pltpu.VMEM((BS, 2 * BS), jnp.int32),
        pltpu.VMEM((2 * M_PAD,), jnp.float32),
        pltpu.VMEM((2 * M_PAD,), jnp.float32),
        pltpu.VMEM((BS, 2 * BS), jnp.float32),
        pltpu.VMEM((BS, 2 * BS), jnp.float32),
        pltpu.SemaphoreType.DMA,
        pltpu.SemaphoreType.DMA,
        pltpu.SemaphoreType.DMA,
        pltpu.SemaphoreType.DMA,
    ],
    compiler_params=pltpu.CompilerParams(needs_layout_passes=False),
)
def _gather_kernel(scores_hbm, info_hbm, out_hbm,
                   info0, info1, rows0, rows1, out0, out1,
                   si0, si1, so0, so1):
    wid = lax.axis_index("s") * NC + lax.axis_index("c")
    t0 = wid * TASKS_PER_W
    slots = ((info0, rows0, out0, si0, so0),
             (info1, rows1, out1, si1, so1))

    def issue_in(t, s):
        inf, rows, _, si, _ = slots[s]
        brp = t // BS
        i = lax.rem(t, BS)
        pltpu.async_copy(
            info_hbm.at[pl.ds(i * BS, BS), pl.ds(brp * 2 * BS, 2 * BS)],
            inf, si)
        pltpu.async_copy(scores_hbm.at[brp * 2 * BS + i],
                         rows.at[pl.ds(0, M_PAD)], si)
        pltpu.async_copy(scores_hbm.at[brp * 2 * BS + BS + i],
                         rows.at[pl.ds(M_PAD, M_PAD)], si)

    def wait_in(s):
        inf, rows, _, si, _ = slots[s]
        pltpu.make_async_copy(
            info_hbm.at[pl.ds(0, BS), pl.ds(0, 2 * BS)], inf, si).wait()
        pltpu.make_async_copy(scores_hbm.at[0], rows.at[pl.ds(0, M_PAD)],
                              si).wait()
        pltpu.make_async_copy(scores_hbm.at[0], rows.at[pl.ds(M_PAD, M_PAD)],
                              si).wait()

    def compute(s):
        inf, rows, out, _, _ = slots[s]

        @plsc.parallel_loop(0, BS, unroll=4)
        def do_j(j):
            for k in range(2 * BS // 16):
                iv = inf[j, pl.ds(k * 16, 16)]
                if k >= BS // 16:
                    iv = iv + M_PAD
                out[j, pl.ds(k * 16, 16)] = plsc.load_gather(rows, [iv])

    def issue_out(t, s):
        _, _, out, _, so = slots[s]
        brp = t // BS
        i = lax.rem(t, BS)
        pltpu.async_copy(
            out,
            out_hbm.at[pl.ds(i * BS, BS), pl.ds(brp * 2 * BS, 2 * BS)], so)

    def wait_out(s):
        _, _, out, _, so = slots[s]
        pltpu.make_async_copy(
            out, out_hbm.at[pl.ds(0, BS), pl.ds(0, 2 * BS)], so).wait()

    issue_in(t0, 0)
    issue_in(t0 + 1, 1)

    def body(t2, carry):
        t = t0 + 2 * t2
        for s in range(2):
            wait_in(s)

            @pl.when(t2 > 0)
            def _():
                wait_out(s)

            compute(s)
            issue_out(t + s, s)

            @pl.when(t2 < TASKS_PER_W // 2 - 1)
            def _():
                issue_in(t + 2 + s, s)
        return carry

    lax.fori_loop(0, TASKS_PER_W // 2, body, 0)
    wait_out(0)
    wait_out(1)


def kernel(q, emb, info, sparsity_layout):
    del sparsity_layout
    emb_p = jnp.pad(emb, ((0, 0), (0, M_PAD - M_EMB), (0, 0)))
    scores = _scores(q.astype(jnp.bfloat16),
                     emb_p.astype(jnp.bfloat16)).reshape(B * S, M_PAD)
    info_v = info.transpose(1, 2, 0).reshape(BS * BS, N_BLK)
    out_v = _gather_kernel(scores, info_v)
    return out_v.reshape(BS, BS, N_BLK).transpose(2, 0, 1)
